# Initial kernel scaffold; baseline (speedup 1.0000x reference)
import functools
import math

import jax
import jax.numpy as jnp
from jax import lax
from jax.experimental import pallas as pl
from jax.experimental.pallas import tpu as pltpu
from jax.experimental.pallas import tpu_sc as plsc

N_NODES = 10000
N_EDGES = 320000
H = 128

NC = 2
NS = 16
NW = NC * NS
EPW = N_EDGES // NW
C = 80
NCHUNK = EPW // C
ROWS_PER_TILE = N_NODES // NS

_f32 = jnp.float32


def _sc_mesh():
    return plsc.VectorSubcoreMesh(
        core_axis_name="c", subcore_axis_name="s", num_cores=NC, num_subcores=NS
    )


def _gather_kernel(hv_hbm, src_hbm, dst_hbm, gs_out, gd_out,
                   idx_s, idx_d, buf_s, buf_d, sem_s, sem_d):
    c = lax.axis_index("c")
    s = lax.axis_index("s")
    wid = s * NC + c
    pltpu.sync_copy(src_hbm.at[wid], idx_s)
    pltpu.sync_copy(dst_hbm.at[wid], idx_d)

    def body(i, carry):
        off = wid * EPW + i * C
        a = pltpu.async_copy(hv_hbm.at[idx_s.at[i]], buf_s, sem_s)
        b = pltpu.async_copy(hv_hbm.at[idx_d.at[i]], buf_d, sem_d)
        a.wait()
        pltpu.sync_copy(buf_s, gs_out.at[pl.ds(off, C)])
        b.wait()
        pltpu.sync_copy(buf_d, gd_out.at[pl.ds(off, C)])
        return carry

    lax.fori_loop(0, NCHUNK, body, 0)


def _sc_gather(h_v, src_rs, dst_rs):
    k = functools.partial(
        pl.kernel,
        out_type=[
            jax.ShapeDtypeStruct((N_EDGES, H), _f32),
            jax.ShapeDtypeStruct((N_EDGES, H), _f32),
        ],
        mesh=_sc_mesh(),
        scratch_types=[
            pltpu.VMEM((NCHUNK, C), jnp.int32),
            pltpu.VMEM((NCHUNK, C), jnp.int32),
            pltpu.VMEM((C, H), _f32),
            pltpu.VMEM((C, H), _f32),
            pltpu.SemaphoreType.DMA,
            pltpu.SemaphoreType.DMA,
        ],
    )(_gather_kernel)
    return k(h_v, src_rs, dst_rs)


def _scatter_kernel(p_hbm, ew_hbm, src_hbm, z128_hbm, z16_hbm, num_out, den_out,
                    idx_s, pbuf, ebuf, acc_n, acc_d):
    c = lax.axis_index("c")
    s = lax.axis_index("s")
    wid = s * NC + c
    pltpu.sync_copy(src_hbm.at[wid], idx_s)

    @pl.when(s == 0)
    def _():
        pltpu.sync_copy(z128_hbm, acc_n)
        pltpu.sync_copy(z16_hbm, acc_d)

    plsc.subcore_barrier()

    def body(i, carry):
        off = wid * EPW + i * C
        pltpu.sync_copy(p_hbm.at[pl.ds(off, C)], pbuf)
        pltpu.sync_copy(ew_hbm.at[pl.ds(off, C)], ebuf)
        pltpu.sync_copy(pbuf, acc_n.at[idx_s.at[i]], add=True)
        pltpu.sync_copy(ebuf, acc_d.at[idx_s.at[i]], add=True)
        return carry

    lax.fori_loop(0, NCHUNK, body, 0)
    plsc.subcore_barrier()

    r0 = s * ROWS_PER_TILE
    pltpu.sync_copy(acc_n.at[pl.ds(r0, ROWS_PER_TILE)],
                    num_out.at[pl.ds(c * N_NODES + r0, ROWS_PER_TILE)])
    pltpu.sync_copy(acc_d.at[pl.ds(r0, ROWS_PER_TILE)],
                    den_out.at[pl.ds(c * N_NODES + r0, ROWS_PER_TILE)])


def _sc_scatter(p, ew, src_rs):
    z128 = jnp.zeros((N_NODES, H), _f32)
    z16 = jnp.zeros((N_NODES, 16), _f32)
    k = functools.partial(
        pl.kernel,
        out_type=[
            jax.ShapeDtypeStruct((NC * N_NODES, H), _f32),
            jax.ShapeDtypeStruct((NC * N_NODES, 16), _f32),
        ],
        mesh=_sc_mesh(),
        scratch_types=[
            pltpu.VMEM((NCHUNK, C), jnp.int32),
            pltpu.VMEM((C, H), _f32),
            pltpu.VMEM((C, 16), _f32),
            pltpu.VMEM_SHARED((N_NODES, H), _f32),
            pltpu.VMEM_SHARED((N_NODES, 16), _f32),
        ],
    )(_scatter_kernel)
    return k(p, ew, src_rs, z128, z16)


B_TC = 2000
GRID_E = N_EDGES // B_TC


def _attn_edge_kernel(hE, gs, gd, wb1a, wb1b, wb1c, bb1, wb2, bb2, wb3t, bb3,
                      wv1a, wv1b, bv1, wv2, bv2, wv3, bv3, p_o, ew_o):
    x = (jnp.dot(gs[...], wb1a[...], preferred_element_type=_f32)
         + jnp.dot(hE[...], wb1b[...], preferred_element_type=_f32)
         + jnp.dot(gd[...], wb1c[...], preferred_element_type=_f32)
         + bb1[...])
    x = jax.nn.gelu(x)
    x = jax.nn.gelu(jnp.dot(x, wb2[...], preferred_element_type=_f32) + bb2[...])
    w = jnp.sum(x * wb3t[...], axis=1, keepdims=True) + bb3[0, 0]
    w = w * (1.0 / math.sqrt(H))
    v = (jnp.dot(hE[...], wv1a[...], preferred_element_type=_f32)
         + jnp.dot(gd[...], wv1b[...], preferred_element_type=_f32)
         + bv1[...])
    v = jax.nn.gelu(v)
    v = jax.nn.gelu(jnp.dot(v, wv2[...], preferred_element_type=_f32) + bv2[...])
    v = jnp.dot(v, wv3[...], preferred_element_type=_f32) + bv3[...]
    e = jnp.exp(w)
    p_o[...] = v * e
    ew_o[...] = jnp.broadcast_to(e, (B_TC, 16))


def _tc_attn_edges(hE, gs, gd, pa):
    wb1 = pa["Bias"][0]["W"]
    wv1 = pa["W_V"][0]["W"]
    wb3 = pa["Bias"][2]["W"]
    args = (
        hE, gs, gd,
        wb1[:H], wb1[H:2 * H], wb1[2 * H:],
        pa["Bias"][0]["b"].reshape(1, H),
        pa["Bias"][1]["W"], pa["Bias"][1]["b"].reshape(1, H),
        wb3.T, jnp.broadcast_to(pa["Bias"][2]["b"].reshape(1, 1), (1, H)),
        wv1[:H], wv1[H:],
        pa["W_V"][0]["b"].reshape(1, H),
        pa["W_V"][1]["W"], pa["W_V"][1]["b"].reshape(1, H),
        pa["W_V"][2]["W"], pa["W_V"][2]["b"].reshape(1, H),
    )
    blk = lambda r, cdim: pl.BlockSpec((r, cdim), lambda i: (i, 0))
    cst = lambda r, cdim: pl.BlockSpec((r, cdim), lambda i: (0, 0))
    in_specs = [blk(B_TC, H)] * 3 + [
        cst(H, H), cst(H, H), cst(H, H), cst(1, H),
        cst(H, H), cst(1, H),
        cst(1, H), cst(1, H),
        cst(H, H), cst(H, H), cst(1, H),
        cst(H, H), cst(1, H),
        cst(H, H), cst(1, H),
    ]
    return pl.pallas_call(
        _attn_edge_kernel,
        grid=(GRID_E,),
        in_specs=in_specs,
        out_specs=[blk(B_TC, H), blk(B_TC, 16)],
        out_shape=[
            jax.ShapeDtypeStruct((N_EDGES, H), _f32),
            jax.ShapeDtypeStruct((N_EDGES, 16), _f32),
        ],
    )(*args)


def _node_kernel(num2, den2, hv, wo, g0, be0, wd0, bd0, wd1, bd1, g1, be1, out):
    num = num2[:N_NODES] + num2[N_NODES:]
    den = den2[:N_NODES, 0:1] + den2[N_NODES:, 0:1]
    attn = num / (den + 1e-12)
    x = hv[...] + jnp.dot(attn, wo[...], preferred_element_type=_f32)
    mu = jnp.mean(x, axis=0, keepdims=True)
    var = jnp.mean((x - mu) ** 2, axis=0, keepdims=True)
    h1 = (x - mu) / jnp.sqrt(var + 1e-5) * g0[...] + be0[...]
    t = jax.nn.gelu(jnp.dot(h1, wd0[...], preferred_element_type=_f32) + bd0[...])
    x2 = h1 + jnp.dot(t, wd1[...], preferred_element_type=_f32) + bd1[...]
    mu2 = jnp.mean(x2, axis=0, keepdims=True)
    var2 = jnp.mean((x2 - mu2) ** 2, axis=0, keepdims=True)
    out[...] = (x2 - mu2) / jnp.sqrt(var2 + 1e-5) * g1[...] + be1[...]


def _tc_node(num2, den2, hv, params):
    args = (
        num2, den2, hv,
        params["attention"]["W_O"]["W"],
        params["norm0"]["gamma"].reshape(1, H), params["norm0"]["beta"].reshape(1, H),
        params["dense0"]["W"], params["dense0"]["b"].reshape(1, 4 * H),
        params["dense1"]["W"], params["dense1"]["b"].reshape(1, H),
        params["norm1"]["gamma"].reshape(1, H), params["norm1"]["beta"].reshape(1, H),
    )
    return pl.pallas_call(
        _node_kernel,
        out_shape=jax.ShapeDtypeStruct((N_NODES, H), _f32),
    )(*args)


def _edge_mlp_kernel(gs, hE, gd, w1a, w1b, w1c, b1, w2, b2, w3, b3, t_o, st_o):
    i = pl.program_id(0)

    @pl.when(i == 0)
    def _():
        st_o[...] = jnp.zeros_like(st_o)

    x = (jnp.dot(gs[...], w1a[...], preferred_element_type=_f32)
         + jnp.dot(hE[...], w1b[...], preferred_element_type=_f32)
         + jnp.dot(gd[...], w1c[...], preferred_element_type=_f32)
         + b1[...])
    x = jax.nn.gelu(x)
    x = jax.nn.gelu(jnp.dot(x, w2[...], preferred_element_type=_f32) + b2[...])
    x = jnp.dot(x, w3[...], preferred_element_type=_f32) + b3[...]
    t = hE[...] + x
    t_o[...] = t
    st_o[0:1, :] += jnp.sum(t, axis=0, keepdims=True)
    st_o[1:2, :] += jnp.sum(t * t, axis=0, keepdims=True)


def _edge_bn_kernel(t, st, g, be, out):
    mu = st[0:1, :] * (1.0 / N_EDGES)
    var = st[1:2, :] * (1.0 / N_EDGES) - mu * mu
    out[...] = (t[...] - mu) / jnp.sqrt(var + 1e-5) * g[...] + be[...]


def _tc_edge_update(gs2, hE, gd2, pe):
    w11 = pe["W11"]["W"]
    args = (
        gs2, hE, gd2,
        w11[:H], w11[H:2 * H], w11[2 * H:],
        pe["W11"]["b"].reshape(1, H),
        pe["W12"]["W"], pe["W12"]["b"].reshape(1, H),
        pe["W13"]["W"], pe["W13"]["b"].reshape(1, H),
    )
    blk = lambda: pl.BlockSpec((B_TC, H), lambda i: (i, 0))
    cst = lambda r: pl.BlockSpec((r, H), lambda i: (0, 0))
    t, st = pl.pallas_call(
        _edge_mlp_kernel,
        grid=(GRID_E,),
        in_specs=[blk()] * 3 + [cst(H), cst(H), cst(H), cst(1),
                                cst(H), cst(1), cst(H), cst(1)],
        out_specs=[blk(), pl.BlockSpec((2, H), lambda i: (0, 0))],
        out_shape=[
            jax.ShapeDtypeStruct((N_EDGES, H), _f32),
            jax.ShapeDtypeStruct((2, H), _f32),
        ],
    )(*args)
    return pl.pallas_call(
        _edge_bn_kernel,
        grid=(GRID_E,),
        in_specs=[blk(), pl.BlockSpec((2, H), lambda i: (0, 0)),
                  cst(1), cst(1)],
        out_specs=blk(),
        out_shape=jax.ShapeDtypeStruct((N_EDGES, H), _f32),
    )(t, st, pe["norm"]["gamma"].reshape(1, H), pe["norm"]["beta"].reshape(1, H))


def kernel(h_V, h_E, params, edge_idx, batch_id):
    src = edge_idx[0].astype(jnp.int32)
    dst = edge_idx[1].astype(jnp.int32)
    src_rs = src.reshape(NW, NCHUNK, C)
    dst_rs = dst.reshape(NW, NCHUNK, C)

    gs, gd = _sc_gather(h_V, src_rs, dst_rs)
    p, ew = _tc_attn_edges(h_E, gs, gd, params["attention"])
    num2, den2 = _sc_scatter(p, ew, src_rs)
    h_V2 = _tc_node(num2, den2, h_V, params)
    gs2, gd2 = _sc_gather(h_V2, src_rs, dst_rs)
    h_E2 = _tc_edge_update(gs2, h_E, gd2, params["edge"])
    return h_V2, h_E2


# speedup vs baseline: 4.3870x; 4.3870x over previous
import functools
import math

import jax
import jax.numpy as jnp
from jax import lax
from jax.experimental import pallas as pl
from jax.experimental.pallas import tpu as pltpu
from jax.experimental.pallas import tpu_sc as plsc

N_NODES = 10000
N_EDGES = 320000
H = 128

NC = 2
NS = 16
NW = NC * NS
EPW = N_EDGES // NW
C = 80
NCHUNK = EPW // C
CS = 128
NCHS = 80
EPWS = NCHS * CS
EP = NW * EPWS
N_PAD = 10240
ROWS_T = N_PAD // NS
WCHUNK = 80

_f32 = jnp.float32


def _sc_mesh():
    return plsc.VectorSubcoreMesh(
        core_axis_name="c", subcore_axis_name="s", num_cores=NC, num_subcores=NS
    )


def _gather_kernel(hv_hbm, src_hbm, dst_hbm, gs_out, gd_out,
                   idx_s, idx_d, buf_s, buf_d, sem_s, sem_d):
    c = lax.axis_index("c")
    s = lax.axis_index("s")
    wid = s * NC + c
    pltpu.sync_copy(src_hbm.at[wid], idx_s)
    pltpu.sync_copy(dst_hbm.at[wid], idx_d)

    def body(i, carry):
        off = wid * EPW + i * C
        a = pltpu.async_copy(hv_hbm.at[idx_s.at[i]], buf_s, sem_s)
        b = pltpu.async_copy(hv_hbm.at[idx_d.at[i]], buf_d, sem_d)
        a.wait()
        pltpu.sync_copy(buf_s, gs_out.at[pl.ds(off, C)])
        b.wait()
        pltpu.sync_copy(buf_d, gd_out.at[pl.ds(off, C)])
        return carry

    lax.fori_loop(0, NCHUNK, body, 0)


def _sc_gather(h_v, src_rs, dst_rs):
    k = functools.partial(
        pl.kernel,
        out_type=[
            jax.ShapeDtypeStruct((N_EDGES, H), _f32),
            jax.ShapeDtypeStruct((N_EDGES, H), _f32),
        ],
        mesh=_sc_mesh(),
        scratch_types=[
            pltpu.VMEM((NCHUNK, C), jnp.int32),
            pltpu.VMEM((NCHUNK, C), jnp.int32),
            pltpu.VMEM((C, H), _f32),
            pltpu.VMEM((C, H), _f32),
            pltpu.SemaphoreType.DMA,
            pltpu.SemaphoreType.DMA,
        ],
    )(_gather_kernel)
    return k(h_v, src_rs, dst_rs)


def _make_scatter_kernel(width):
    def _scatter_kernel(p_hbm, src_hbm, z_hbm, out, idx_s, pbuf, acc):
        c = lax.axis_index("c")
        s = lax.axis_index("s")
        wid = s * NC + c
        pltpu.sync_copy(src_hbm.at[pl.ds(wid * NCHS, NCHS)], idx_s)

        r0 = s * ROWS_T

        def zbody(j, carry):
            off = r0 + j * WCHUNK
            pltpu.sync_copy(z_hbm.at[pl.ds(off, WCHUNK)], pbuf.at[pl.ds(0, WCHUNK)])
            pltpu.sync_copy(pbuf.at[pl.ds(0, WCHUNK)], acc.at[pl.ds(off, WCHUNK)])
            return carry

        lax.fori_loop(0, ROWS_T // WCHUNK, zbody, 0)
        plsc.subcore_barrier()

        def body(i, carry):
            off = wid * EPWS + i * CS
            pltpu.sync_copy(p_hbm.at[pl.ds(off, CS)], pbuf)
            pltpu.sync_copy(pbuf, acc.at[idx_s.at[i]], add=True)
            return carry

        lax.fori_loop(0, NCHS, body, 0)
        plsc.subcore_barrier()

        def wbody(j, carry):
            off = r0 + j * WCHUNK
            pltpu.sync_copy(acc.at[pl.ds(off, WCHUNK)], pbuf.at[pl.ds(0, WCHUNK)])
            pltpu.sync_copy(pbuf.at[pl.ds(0, WCHUNK)],
                            out.at[pl.ds(c * N_PAD + off, WCHUNK)])
            return carry

        lax.fori_loop(0, ROWS_T // WCHUNK, wbody, 0)

    return _scatter_kernel


def _sc_scatter(p, width, src_pad2d):
    z = jnp.zeros((N_PAD, width), _f32)
    k = functools.partial(
        pl.kernel,
        out_type=jax.ShapeDtypeStruct((NC * N_PAD, width), _f32),
        mesh=_sc_mesh(),
        scratch_types=[
            pltpu.VMEM((NCHS, CS), jnp.int32),
            pltpu.VMEM((CS, width), _f32),
            pltpu.VMEM_SHARED((N_PAD, width), _f32),
        ],
    )(_make_scatter_kernel(width))
    return k(p, src_pad2d, z)


B_TC = 2560
GRID_E = N_EDGES // B_TC


def _attn_edge_kernel(hE, gs, gd, wb1a, wb1b, wb1c, bb1, wb2, bb2, wb3t, bb3,
                      wv1a, wv1b, bv1, wv2, bv2, wv3, bv3, p_o, ew_o):
    x = (jnp.dot(gs[...], wb1a[...], preferred_element_type=_f32)
         + jnp.dot(hE[...], wb1b[...], preferred_element_type=_f32)
         + jnp.dot(gd[...], wb1c[...], preferred_element_type=_f32)
         + bb1[...])
    x = jax.nn.gelu(x)
    x = jax.nn.gelu(jnp.dot(x, wb2[...], preferred_element_type=_f32) + bb2[...])
    w = jnp.sum(x * wb3t[...], axis=1, keepdims=True) + bb3[0, 0]
    w = w * (1.0 / math.sqrt(H))
    v = (jnp.dot(hE[...], wv1a[...], preferred_element_type=_f32)
         + jnp.dot(gd[...], wv1b[...], preferred_element_type=_f32)
         + bv1[...])
    v = jax.nn.gelu(v)
    v = jax.nn.gelu(jnp.dot(v, wv2[...], preferred_element_type=_f32) + bv2[...])
    v = jnp.dot(v, wv3[...], preferred_element_type=_f32) + bv3[...]
    e = jnp.exp(w)
    p_o[...] = v * e
    ew_o[...] = jnp.broadcast_to(e, (B_TC, H))


def _tc_attn_edges(hE, gs, gd, pa):
    wb1 = pa["Bias"][0]["W"]
    wv1 = pa["W_V"][0]["W"]
    wb3 = pa["Bias"][2]["W"]
    args = (
        hE, gs, gd,
        wb1[:H], wb1[H:2 * H], wb1[2 * H:],
        pa["Bias"][0]["b"].reshape(1, H),
        pa["Bias"][1]["W"], pa["Bias"][1]["b"].reshape(1, H),
        wb3.T, jnp.broadcast_to(pa["Bias"][2]["b"].reshape(1, 1), (1, H)),
        wv1[:H], wv1[H:],
        pa["W_V"][0]["b"].reshape(1, H),
        pa["W_V"][1]["W"], pa["W_V"][1]["b"].reshape(1, H),
        pa["W_V"][2]["W"], pa["W_V"][2]["b"].reshape(1, H),
    )
    blk = lambda r, cdim: pl.BlockSpec((r, cdim), lambda i: (i, 0))
    cst = lambda r, cdim: pl.BlockSpec((r, cdim), lambda i: (0, 0))
    in_specs = [blk(B_TC, H)] * 3 + [
        cst(H, H), cst(H, H), cst(H, H), cst(1, H),
        cst(H, H), cst(1, H),
        cst(1, H), cst(1, H),
        cst(H, H), cst(H, H), cst(1, H),
        cst(H, H), cst(1, H),
        cst(H, H), cst(1, H),
    ]
    return pl.pallas_call(
        _attn_edge_kernel,
        grid=(GRID_E,),
        in_specs=in_specs,
        out_specs=[blk(B_TC, H), blk(B_TC, H)],
        out_shape=[
            jax.ShapeDtypeStruct((EP, H), _f32),
            jax.ShapeDtypeStruct((EP, H), _f32),
        ],
    )(*args)


def _node_kernel(num2, den2, hv, wo, g0, be0, wd0, bd0, wd1, bd1, g1, be1, out):
    num = num2[:N_NODES] + num2[N_PAD:N_PAD + N_NODES]
    den = den2[:N_NODES, 0:1] + den2[N_PAD:N_PAD + N_NODES, 0:1]
    attn = num / (den + 1e-12)
    x = hv[...] + jnp.dot(attn, wo[...], preferred_element_type=_f32)
    mu = jnp.mean(x, axis=0, keepdims=True)
    var = jnp.mean((x - mu) ** 2, axis=0, keepdims=True)
    h1 = (x - mu) / jnp.sqrt(var + 1e-5) * g0[...] + be0[...]
    nch = 5
    rows = N_NODES // nch
    ssum = jnp.zeros((1, H), _f32)
    ssq = jnp.zeros((1, H), _f32)
    for i in range(nch):
        hc = h1[i * rows:(i + 1) * rows]
        t = jax.nn.gelu(jnp.dot(hc, wd0[...], preferred_element_type=_f32) + bd0[...])
        x2 = hc + jnp.dot(t, wd1[...], preferred_element_type=_f32) + bd1[...]
        out[i * rows:(i + 1) * rows, :] = x2
        ssum = ssum + jnp.sum(x2, axis=0, keepdims=True)
        ssq = ssq + jnp.sum(x2 * x2, axis=0, keepdims=True)
    mu2 = ssum * (1.0 / N_NODES)
    var2 = ssq * (1.0 / N_NODES) - mu2 * mu2
    out[...] = (out[...] - mu2) / jnp.sqrt(var2 + 1e-5) * g1[...] + be1[...]


def _tc_node(num2, den2, hv, params):
    args = (
        num2, den2, hv,
        params["attention"]["W_O"]["W"],
        params["norm0"]["gamma"].reshape(1, H), params["norm0"]["beta"].reshape(1, H),
        params["dense0"]["W"], params["dense0"]["b"].reshape(1, 4 * H),
        params["dense1"]["W"], params["dense1"]["b"].reshape(1, H),
        params["norm1"]["gamma"].reshape(1, H), params["norm1"]["beta"].reshape(1, H),
    )
    return pl.pallas_call(
        _node_kernel,
        out_shape=jax.ShapeDtypeStruct((N_NODES, H), _f32),
    )(*args)


def _edge_mlp_kernel(gs, hE, gd, w1a, w1b, w1c, b1, w2, b2, w3, b3, t_o, st_o):
    i = pl.program_id(0)

    @pl.when(i == 0)
    def _():
        st_o[...] = jnp.zeros_like(st_o)

    x = (jnp.dot(gs[...], w1a[...], preferred_element_type=_f32)
         + jnp.dot(hE[...], w1b[...], preferred_element_type=_f32)
         + jnp.dot(gd[...], w1c[...], preferred_element_type=_f32)
         + b1[...])
    x = jax.nn.gelu(x)
    x = jax.nn.gelu(jnp.dot(x, w2[...], preferred_element_type=_f32) + b2[...])
    x = jnp.dot(x, w3[...], preferred_element_type=_f32) + b3[...]
    t = hE[...] + x
    t_o[...] = t
    st_o[0:1, :] += jnp.sum(t, axis=0, keepdims=True)
    st_o[1:2, :] += jnp.sum(t * t, axis=0, keepdims=True)


def _edge_bn_kernel(t, st, g, be, out):
    mu = st[0:1, :] * (1.0 / N_EDGES)
    var = st[1:2, :] * (1.0 / N_EDGES) - mu * mu
    out[...] = (t[...] - mu) / jnp.sqrt(var + 1e-5) * g[...] + be[...]


def _tc_edge_update(gs2, hE, gd2, pe):
    w11 = pe["W11"]["W"]
    args = (
        gs2, hE, gd2,
        w11[:H], w11[H:2 * H], w11[2 * H:],
        pe["W11"]["b"].reshape(1, H),
        pe["W12"]["W"], pe["W12"]["b"].reshape(1, H),
        pe["W13"]["W"], pe["W13"]["b"].reshape(1, H),
    )
    blk = lambda: pl.BlockSpec((B_TC, H), lambda i: (i, 0))
    cst = lambda r: pl.BlockSpec((r, H), lambda i: (0, 0))
    t, st = pl.pallas_call(
        _edge_mlp_kernel,
        grid=(GRID_E,),
        in_specs=[blk()] * 3 + [cst(H), cst(H), cst(H), cst(1),
                                cst(H), cst(1), cst(H), cst(1)],
        out_specs=[blk(), pl.BlockSpec((2, H), lambda i: (0, 0))],
        out_shape=[
            jax.ShapeDtypeStruct((N_EDGES, H), _f32),
            jax.ShapeDtypeStruct((2, H), _f32),
        ],
    )(*args)
    return pl.pallas_call(
        _edge_bn_kernel,
        grid=(GRID_E,),
        in_specs=[blk(), pl.BlockSpec((2, H), lambda i: (0, 0)),
                  cst(1), cst(1)],
        out_specs=blk(),
        out_shape=jax.ShapeDtypeStruct((N_EDGES, H), _f32),
    )(t, st, pe["norm"]["gamma"].reshape(1, H), pe["norm"]["beta"].reshape(1, H))


def kernel(h_V, h_E, params, edge_idx, batch_id):
    src = edge_idx[0].astype(jnp.int32)
    dst = edge_idx[1].astype(jnp.int32)
    src_rs = src.reshape(NW, NCHUNK, C)
    dst_rs = dst.reshape(NW, NCHUNK, C)
    src_pad2d = jnp.concatenate(
        [src, jnp.full((EP - N_EDGES,), N_NODES, jnp.int32)]).reshape(EP // CS, CS)

    gs, gd = _sc_gather(h_V, src_rs, dst_rs)
    p, ew = _tc_attn_edges(h_E, gs, gd, params["attention"])
    num2 = _sc_scatter(p, H, src_pad2d)
    den2 = _sc_scatter(ew, H, src_pad2d)
    h_V2 = _tc_node(num2, den2, h_V, params)
    gs2, gd2 = _sc_gather(h_V2, src_rs, dst_rs)
    h_E2 = _tc_edge_update(gs2, h_E, gd2, params["edge"])
    return h_V2, h_E2
